# baseline (speedup 1.0000x reference)
import functools

import jax
import jax.numpy as jnp
from jax import lax
from jax.experimental import pallas as pl
from jax.experimental.pallas import tpu as pltpu
from jax.experimental.pallas import tpu_sc as plsc

B, A, H, W = 4, 8, 384, 384
HW = H * W
NG = 4096
NPT = NG // 8
PPT = HW // 8
CS = 2304
RPC = CS // W
NSUB = PPT // CS
GRP = CS // 16
SCALE = -1.0 / (0.075 * 0.075 * 2.0)


def _sc_body(x_hbm, vids_hbm, gni_hbm, out_hbm,
             jv, gbuf, spm, tbx, tby, tbz, vv, ov, outv,
             gsem, sin0, sin1, sout0, sout1):
    c = lax.axis_index("c")
    s = lax.axis_index("s")
    b = 2 * c + s // 8
    t = s % 8
    bb = s // 8

    def in_descs(chunk, buf, sem):
        goff = t * PPT + chunk * CS
        ds_ = []
        for a in range(A):
            ds_.append(pltpu.make_async_copy(
                vids_hbm.at[b, a, pl.ds(goff, CS)],
                vv.at[pl.ds((buf * A + a) * CS, CS)], sem))
        for ci in range(3):
            ds_.append(pltpu.make_async_copy(
                x_hbm.at[b, ci, pl.ds(goff, CS)],
                ov.at[pl.ds((buf * 3 + ci) * CS, CS)], sem))
        return ds_

    def out_descs(chunk, buf, sem):
        goff = t * PPT + chunk * CS
        r0 = goff // W
        ds_ = []
        for a in range(A):
            for rr in range(RPC):
                ds_.append(pltpu.make_async_copy(
                    outv.at[pl.ds((buf * A + a) * CS + rr * W, W)],
                    out_hbm.at[b, a, r0 + rr], sem))
        return ds_

    for d in in_descs(0, 0, sin0):
        d.start()

    pltpu.sync_copy(gni_hbm.at[b, pl.ds(t * NPT, NPT)], jv)
    descs = []
    for k in range(NPT // 128):
        idx = jv.at[pl.ds(k * 128, 128)]
        for ci in range(3):
            descs.append(pltpu.async_copy(
                x_hbm.at[b, ci].at[idx],
                gbuf.at[pl.ds(ci * NPT + k * 128, 128)], gsem))
    for d in descs:
        d.wait()
    for ci in range(3):
        pltpu.sync_copy(gbuf.at[pl.ds(ci * NPT, NPT)],
                        spm.at[pl.ds(bb * 3 * NG + ci * NG + t * NPT, NPT)])
    plsc.subcore_barrier()
    pltpu.sync_copy(spm.at[pl.ds(bb * 3 * NG + 0 * NG, NG)], tbx)
    pltpu.sync_copy(spm.at[pl.ds(bb * 3 * NG + 1 * NG, NG)], tby)
    pltpu.sync_copy(spm.at[pl.ds(bb * 3 * NG + 2 * NG, NG)], tbz)

    def compute_chunk(buf):
        vb = buf * A * CS
        ob = buf * 3 * CS
        wb = buf * A * CS

        @plsc.parallel_loop(0, GRP, 1, unroll=4)
        def _(i):
            o16 = i * 16
            ox = ov[pl.ds(ob + 0 * CS + o16, 16)]
            oy = ov[pl.ds(ob + 1 * CS + o16, 16)]
            oz = ov[pl.ds(ob + 2 * CS + o16, 16)]
            d2 = []
            for a in range(A):
                vid = vv[pl.ds(vb + a * CS + o16, 16)]
                px = plsc.load_gather(tbx, [vid])
                py = plsc.load_gather(tby, [vid])
                pz = plsc.load_gather(tbz, [vid])
                dx = ox - px
                dy = oy - py
                dz = oz - pz
                d2.append(dx * dx + dy * dy + dz * dz)
            m01 = jnp.minimum(d2[0], d2[1])
            m23 = jnp.minimum(d2[2], d2[3])
            m45 = jnp.minimum(d2[4], d2[5])
            m67 = jnp.minimum(d2[6], d2[7])
            mn = jnp.minimum(jnp.minimum(m01, m23), jnp.minimum(m45, m67))
            es = [jnp.exp((d - mn) * SCALE) for d in d2]
            ssum = ((es[0] + es[1]) + (es[2] + es[3])) + \
                   ((es[4] + es[5]) + (es[6] + es[7]))
            inv = 1.0 / ssum
            for a in range(A):
                outv[pl.ds(wb + a * CS + o16, 16)] = es[a] * inv

    def pair_body(k, carry):
        c0 = 2 * k
        c1 = 2 * k + 1
        for d in in_descs(c0, 0, sin0):
            d.wait()
        for d in in_descs(c1, 1, sin1):
            d.start()

        @pl.when(k > 0)
        def _():
            for d in out_descs(c0, 0, sout0):
                d.wait()

        compute_chunk(0)
        for d in out_descs(c0, 0, sout0):
            d.start()

        for d in in_descs(c1, 1, sin1):
            d.wait()

        @pl.when(c1 + 1 < NSUB)
        def _():
            for d in in_descs(c1 + 1, 0, sin0):
                d.start()

        @pl.when(k > 0)
        def _():
            for d in out_descs(c1, 1, sout1):
                d.wait()

        compute_chunk(1)
        for d in out_descs(c1, 1, sout1):
            d.start()
        return carry

    lax.fori_loop(0, NSUB // 2, pair_body, 0)
    for d in out_descs(NSUB - 2, 0, sout0):
        d.wait()
    for d in out_descs(NSUB - 1, 1, sout1):
        d.wait()


@jax.jit
def _blend_skin_sc(x, v_ids, gni):
    x_r = jnp.take(x, jnp.arange(3, 6), axis=1).reshape(B, 3, HW)
    vids_r = v_ids.reshape(B, A, HW)
    mesh = plsc.VectorSubcoreMesh(core_axis_name="c", subcore_axis_name="s")
    run = functools.partial(
        pl.kernel,
        out_type=jax.ShapeDtypeStruct((B, A, H, W), jnp.float32),
        mesh=mesh,
        compiler_params=pltpu.CompilerParams(
            needs_layout_passes=False, use_tc_tiling_on_sc=False),
        scratch_types=[
            pltpu.VMEM((NPT,), jnp.int32),
            pltpu.VMEM((3 * NPT,), jnp.float32),
            pltpu.VMEM_SHARED((2 * 3 * NG,), jnp.float32),
            pltpu.VMEM((NG,), jnp.float32),
            pltpu.VMEM((NG,), jnp.float32),
            pltpu.VMEM((NG,), jnp.float32),
            pltpu.VMEM((2 * A * CS,), jnp.int32),
            pltpu.VMEM((2 * 3 * CS,), jnp.float32),
            pltpu.VMEM((2 * A * CS,), jnp.float32),
            pltpu.SemaphoreType.DMA,
            pltpu.SemaphoreType.DMA,
            pltpu.SemaphoreType.DMA,
            pltpu.SemaphoreType.DMA,
            pltpu.SemaphoreType.DMA,
        ],
    )(_sc_body)
    return run(x_r, vids_r, gni)


def kernel(x, mask, v_ids, Graph_nodes_ids, nodes_mask, Graph_Edge,
           edges_mask, points):
    return _blend_skin_sc(x, v_ids, Graph_nodes_ids)


# speedup vs baseline: 1.7489x; 1.7489x over previous
import functools

import jax
import jax.numpy as jnp
from jax import lax
from jax.experimental import pallas as pl
from jax.experimental.pallas import tpu as pltpu
from jax.experimental.pallas import tpu_sc as plsc

B, A, H, W = 4, 8, 384, 384
HW = H * W
NG = 4096
NPT = NG // 8
PPT = HW // 8
CS = 2304
RPC = CS // W
NSUB = PPT // CS
GRP = CS // 16
SCALE = -1.0 / (0.075 * 0.075 * 2.0)


def _sc_body(x_hbm, vids_hbm, gni_hbm, out_hbm,
             jv, gbuf, spm, tbx, tby, tbz, vv, ov, outv,
             gsem, sin0, sin1, sout0, sout1):
    c = lax.axis_index("c")
    s = lax.axis_index("s")
    b = 2 * c + s // 8
    t = s % 8
    bb = s // 8

    def in_descs(chunk, buf, sem):
        goff = t * PPT + chunk * CS
        ds_ = []
        for a in range(A):
            ds_.append(pltpu.make_async_copy(
                vids_hbm.at[b, a, pl.ds(goff, CS)],
                vv.at[pl.ds((buf * A + a) * CS, CS)], sem))
        for ci in range(3):
            ds_.append(pltpu.make_async_copy(
                x_hbm.at[b, ci, pl.ds(goff, CS)],
                ov.at[pl.ds((buf * 3 + ci) * CS, CS)], sem))
        return ds_

    def out_descs(chunk, buf, sem):
        goff = t * PPT + chunk * CS
        r0 = goff // W
        ds_ = []
        for a in range(A):
            for rr in range(RPC):
                ds_.append(pltpu.make_async_copy(
                    outv.at[pl.ds((buf * A + a) * CS + rr * W, W)],
                    out_hbm.at[b, a, r0 + rr], sem))
        return ds_

    for d in in_descs(0, 0, sin0):
        d.start()

    pltpu.sync_copy(gni_hbm.at[b, pl.ds(t * NPT, NPT)], jv)
    descs = []
    for k in range(NPT // 128):
        idx = jv.at[pl.ds(k * 128, 128)]
        for ci in range(3):
            descs.append(pltpu.async_copy(
                x_hbm.at[b, ci].at[idx],
                gbuf.at[pl.ds(ci * NPT + k * 128, 128)], gsem))
    for d in descs:
        d.wait()
    for ci in range(3):
        pltpu.sync_copy(gbuf.at[pl.ds(ci * NPT, NPT)],
                        spm.at[pl.ds(bb * 3 * NG + ci * NG + t * NPT, NPT)])
    plsc.subcore_barrier()
    pltpu.sync_copy(spm.at[pl.ds(bb * 3 * NG + 0 * NG, NG)], tbx)
    pltpu.sync_copy(spm.at[pl.ds(bb * 3 * NG + 1 * NG, NG)], tby)
    pltpu.sync_copy(spm.at[pl.ds(bb * 3 * NG + 2 * NG, NG)], tbz)

    def compute_chunk(buf):
        vb = buf * A * CS
        ob = buf * 3 * CS
        wb = buf * A * CS

        @plsc.parallel_loop(0, GRP, 1, unroll=2)
        def _(i):
            o16 = i * 16
            ox = ov[pl.ds(ob + 0 * CS + o16, 16)]
            oy = ov[pl.ds(ob + 1 * CS + o16, 16)]
            oz = ov[pl.ds(ob + 2 * CS + o16, 16)]
            d2 = []
            for a in range(A):
                vid = vv[pl.ds(vb + a * CS + o16, 16)]
                px = plsc.load_gather(tbx, [vid])
                py = plsc.load_gather(tby, [vid])
                pz = plsc.load_gather(tbz, [vid])
                dx = ox - px
                dy = oy - py
                dz = oz - pz
                d2.append(dx * dx + dy * dy + dz * dz)
            m01 = jnp.minimum(d2[0], d2[1])
            m23 = jnp.minimum(d2[2], d2[3])
            m45 = jnp.minimum(d2[4], d2[5])
            m67 = jnp.minimum(d2[6], d2[7])
            mn = jnp.minimum(jnp.minimum(m01, m23), jnp.minimum(m45, m67))
            es = [jnp.exp((d - mn) * SCALE) for d in d2]
            ssum = ((es[0] + es[1]) + (es[2] + es[3])) + \
                   ((es[4] + es[5]) + (es[6] + es[7]))
            inv = 1.0 / ssum
            for a in range(A):
                outv[pl.ds(wb + a * CS + o16, 16)] = es[a] * inv

    def pair_body(k, carry):
        c0 = 2 * k
        c1 = 2 * k + 1
        for d in in_descs(c0, 0, sin0):
            d.wait()
        for d in in_descs(c1, 1, sin1):
            d.start()

        @pl.when(k > 0)
        def _():
            for d in out_descs(c0, 0, sout0):
                d.wait()

        compute_chunk(0)
        for d in out_descs(c0, 0, sout0):
            d.start()

        for d in in_descs(c1, 1, sin1):
            d.wait()

        @pl.when(c1 + 1 < NSUB)
        def _():
            for d in in_descs(c1 + 1, 0, sin0):
                d.start()

        @pl.when(k > 0)
        def _():
            for d in out_descs(c1, 1, sout1):
                d.wait()

        compute_chunk(1)
        for d in out_descs(c1, 1, sout1):
            d.start()
        return carry

    lax.fori_loop(0, NSUB // 2, pair_body, 0)
    for d in out_descs(NSUB - 2, 0, sout0):
        d.wait()
    for d in out_descs(NSUB - 1, 1, sout1):
        d.wait()


@jax.jit
def _blend_skin_sc(x, v_ids, gni):
    x_r = x[:, 3:6].reshape(B, 3, HW)
    vids_r = v_ids.reshape(B, A, HW)
    mesh = plsc.VectorSubcoreMesh(core_axis_name="c", subcore_axis_name="s")
    run = functools.partial(
        pl.kernel,
        out_type=jax.ShapeDtypeStruct((B, A, H, W), jnp.float32),
        mesh=mesh,
        compiler_params=pltpu.CompilerParams(
            needs_layout_passes=False, use_tc_tiling_on_sc=False),
        scratch_types=[
            pltpu.VMEM((NPT,), jnp.int32),
            pltpu.VMEM((3 * NPT,), jnp.float32),
            pltpu.VMEM_SHARED((2 * 3 * NG,), jnp.float32),
            pltpu.VMEM((NG,), jnp.float32),
            pltpu.VMEM((NG,), jnp.float32),
            pltpu.VMEM((NG,), jnp.float32),
            pltpu.VMEM((2 * A * CS,), jnp.int32),
            pltpu.VMEM((2 * 3 * CS,), jnp.float32),
            pltpu.VMEM((2 * A * CS,), jnp.float32),
            pltpu.SemaphoreType.DMA,
            pltpu.SemaphoreType.DMA,
            pltpu.SemaphoreType.DMA,
            pltpu.SemaphoreType.DMA,
            pltpu.SemaphoreType.DMA,
        ],
    )(_sc_body)
    return run(x_r, vids_r, gni)


def kernel(x, mask, v_ids, Graph_nodes_ids, nodes_mask, Graph_Edge,
           edges_mask, points):
    return _blend_skin_sc(x, v_ids, Graph_nodes_ids)


# speedup vs baseline: 1.8093x; 1.0346x over previous
import functools

import jax
import jax.numpy as jnp
from jax import lax
from jax.experimental import pallas as pl
from jax.experimental.pallas import tpu as pltpu
from jax.experimental.pallas import tpu_sc as plsc

B, A, H, W = 4, 8, 384, 384
HW = H * W
NG = 4096
NPT = NG // 8
PPT = HW // 8
CS = 2304
RPC = CS // W
NSUB = PPT // CS
GRP = CS // 16
SCALE = -1.0 / (0.075 * 0.075 * 2.0)


def _sc_body(x_hbm, vids_hbm, gni_hbm, out_hbm,
             jv, gbuf, spm, tbx, tby, tbz, vv, ov, outv,
             gsem, sin0, sin1, sout0, sout1):
    c = lax.axis_index("c")
    s = lax.axis_index("s")
    b = 2 * c + s // 8
    t = s % 8
    bb = s // 8

    def in_descs(chunk, buf, sem):
        goff = t * PPT + chunk * CS
        ds_ = []
        for a in range(A):
            ds_.append(pltpu.make_async_copy(
                vids_hbm.at[b, a, pl.ds(goff, CS)],
                vv.at[pl.ds((buf * A + a) * CS, CS)], sem))
        for ci in range(3):
            ds_.append(pltpu.make_async_copy(
                x_hbm.at[b, ci, pl.ds(goff, CS)],
                ov.at[pl.ds((buf * 3 + ci) * CS, CS)], sem))
        return ds_

    def out_descs(chunk, buf, sem):
        goff = t * PPT + chunk * CS
        r0 = goff // W
        ds_ = []
        for a in range(A):
            for rr in range(RPC):
                ds_.append(pltpu.make_async_copy(
                    outv.at[pl.ds((buf * A + a) * CS + rr * W, W)],
                    out_hbm.at[b, a, r0 + rr], sem))
        return ds_

    for d in in_descs(0, 0, sin0):
        d.start()

    pltpu.sync_copy(gni_hbm.at[b, pl.ds(t * NPT, NPT)], jv)
    descs = []
    for k in range(NPT // 128):
        idx = jv.at[pl.ds(k * 128, 128)]
        for ci in range(3):
            descs.append(pltpu.async_copy(
                x_hbm.at[b, ci].at[idx],
                gbuf.at[pl.ds(ci * NPT + k * 128, 128)], gsem))
    for d in descs:
        d.wait()
    for ci in range(3):
        pltpu.sync_copy(gbuf.at[pl.ds(ci * NPT, NPT)],
                        spm.at[pl.ds(bb * 3 * NG + ci * NG + t * NPT, NPT)])
    plsc.subcore_barrier()
    pltpu.sync_copy(spm.at[pl.ds(bb * 3 * NG + 0 * NG, NG)], tbx)
    pltpu.sync_copy(spm.at[pl.ds(bb * 3 * NG + 1 * NG, NG)], tby)
    pltpu.sync_copy(spm.at[pl.ds(bb * 3 * NG + 2 * NG, NG)], tbz)

    def compute_chunk(buf):
        vb = buf * A * CS
        ob = buf * 3 * CS
        wb = buf * A * CS

        @plsc.parallel_loop(0, GRP, 1, unroll=2)
        def _(i):
            o16 = i * 16
            ox = ov[pl.ds(ob + 0 * CS + o16, 16)]
            oy = ov[pl.ds(ob + 1 * CS + o16, 16)]
            oz = ov[pl.ds(ob + 2 * CS + o16, 16)]
            d2 = []
            for a in range(A):
                vid = vv[pl.ds(vb + a * CS + o16, 16)]
                px = plsc.load_gather(tbx, [vid])
                py = plsc.load_gather(tby, [vid])
                pz = plsc.load_gather(tbz, [vid])
                dx = ox - px
                dy = oy - py
                dz = oz - pz
                d2.append(dx * dx + dy * dy + dz * dz)
            m01 = jnp.minimum(d2[0], d2[1])
            m23 = jnp.minimum(d2[2], d2[3])
            m45 = jnp.minimum(d2[4], d2[5])
            m67 = jnp.minimum(d2[6], d2[7])
            mn = jnp.minimum(jnp.minimum(m01, m23), jnp.minimum(m45, m67))
            es = [jnp.exp(mn - d) for d in d2]
            ssum = ((es[0] + es[1]) + (es[2] + es[3])) + \
                   ((es[4] + es[5]) + (es[6] + es[7]))
            inv = 1.0 / ssum
            for a in range(A):
                outv[pl.ds(wb + a * CS + o16, 16)] = es[a] * inv

    def pair_body(k, carry):
        c0 = 2 * k
        c1 = 2 * k + 1
        for d in in_descs(c0, 0, sin0):
            d.wait()
        for d in in_descs(c1, 1, sin1):
            d.start()

        @pl.when(k > 0)
        def _():
            for d in out_descs(c0, 0, sout0):
                d.wait()

        compute_chunk(0)
        for d in out_descs(c0, 0, sout0):
            d.start()

        for d in in_descs(c1, 1, sin1):
            d.wait()

        @pl.when(c1 + 1 < NSUB)
        def _():
            for d in in_descs(c1 + 1, 0, sin0):
                d.start()

        @pl.when(k > 0)
        def _():
            for d in out_descs(c1, 1, sout1):
                d.wait()

        compute_chunk(1)
        for d in out_descs(c1, 1, sout1):
            d.start()
        return carry

    lax.fori_loop(0, NSUB // 2, pair_body, 0)
    for d in out_descs(NSUB - 2, 0, sout0):
        d.wait()
    for d in out_descs(NSUB - 1, 1, sout1):
        d.wait()


@jax.jit
def _blend_skin_sc(x, v_ids, gni):
    x_r = (x[:, 3:6] * (-SCALE) ** 0.5).reshape(B, 3, HW)
    vids_r = v_ids.reshape(B, A, HW)
    mesh = plsc.VectorSubcoreMesh(core_axis_name="c", subcore_axis_name="s")
    run = functools.partial(
        pl.kernel,
        out_type=jax.ShapeDtypeStruct((B, A, H, W), jnp.float32),
        mesh=mesh,
        compiler_params=pltpu.CompilerParams(
            needs_layout_passes=False, use_tc_tiling_on_sc=False),
        scratch_types=[
            pltpu.VMEM((NPT,), jnp.int32),
            pltpu.VMEM((3 * NPT,), jnp.float32),
            pltpu.VMEM_SHARED((2 * 3 * NG,), jnp.float32),
            pltpu.VMEM((NG,), jnp.float32),
            pltpu.VMEM((NG,), jnp.float32),
            pltpu.VMEM((NG,), jnp.float32),
            pltpu.VMEM((2 * A * CS,), jnp.int32),
            pltpu.VMEM((2 * 3 * CS,), jnp.float32),
            pltpu.VMEM((2 * A * CS,), jnp.float32),
            pltpu.SemaphoreType.DMA,
            pltpu.SemaphoreType.DMA,
            pltpu.SemaphoreType.DMA,
            pltpu.SemaphoreType.DMA,
            pltpu.SemaphoreType.DMA,
        ],
    )(_sc_body)
    return run(x_r, vids_r, gni)


def kernel(x, mask, v_ids, Graph_nodes_ids, nodes_mask, Graph_Edge,
           edges_mask, points):
    return _blend_skin_sc(x, v_ids, Graph_nodes_ids)
